# baseline (speedup 1.0000x reference)
import jax
import jax.numpy as jnp
from jax import lax
from jax.experimental import pallas as pl
from jax.experimental.pallas import tpu as pltpu
from jax.experimental.pallas import tpu_sc as plsc

N_NODES = 50000
N_EDGES = 800000
EMB = 64
NC, NS = 2, 16
NW = NC * NS
L = 16

CH = 256
NCH = N_EDGES // CH
NPAIR = 49

NODE_FULL = N_NODES // CH
NODE_TAIL = N_NODES - NODE_FULL * CH
NODE_PAD = 50048

DEG_ROWS = 6256
DEG_GROUPS = DEG_ROWS // 16
HIST_PAD = 50048
DEG_SLICE = 3200


def _sc_body(atom1d, bins1d, bonds1d, src1d, dst1d, dstdeg2d, px, py, pz,
             atab, btab, dtab,
             node_out, dis_out, bond_out, deg_out, dir_out,
             ia0, ib0, is0, id0, ia1, ib1, is1, id1,
             ra0, rb0, ra1, rb1,
             px0, py0, pz0, qx0, qy0, qz0,
             px1, py1, pz1, qx1, qy1, qz1,
             dir0, dir1,
             onesb, dstb, zb, hist_sh,
             semi0, semi1, semg0, semg1, sems0, sems1):
  cid = lax.axis_index("c")
  tid = lax.axis_index("s")
  wid = tid * NC + cid
  iota = lax.iota(jnp.int32, L)

  idx_bufs = ((ia0, ib0, is0, id0), (ia1, ib1, is1, id1))
  rows_bufs = ((ra0, rb0), (ra1, rb1))
  pos_bufs = ((px0, py0, pz0, qx0, qy0, qz0),
              (px1, py1, pz1, qx1, qy1, qz1))
  dir_bufs = (dir0, dir1)
  semi = (semi0, semi1)
  semg = (semg0, semg1)
  sems = (sems0, sems1)
  idx_srcs = (bins1d, bonds1d, src1d, dst1d)

  def fire_idx(c, b):
    base = c * CH
    for buf, src in zip(idx_bufs[b], idx_srcs):
      pltpu.async_copy(src.at[pl.ds(base, CH)], buf, semi[b])

  def drain_idx(b):
    for buf, src in zip(idx_bufs[b], idx_srcs):
      pltpu.make_async_copy(src.at[pl.ds(0, CH)], buf, semi[b]).wait()

  def fire_gat(b):
    ia, ib, isrc, idst = idx_bufs[b]
    ra, rb = rows_bufs[b]
    gx, gy, gz, hx, hy, hz = pos_bufs[b]
    for j in range(CH // 128):
      sl = pl.ds(j * 128, 128)
      pltpu.async_copy(dtab.at[ia.at[sl]], ra.at[sl], semg[b])
      pltpu.async_copy(btab.at[ib.at[sl]], rb.at[sl], semg[b])
      pltpu.async_copy(px.at[idst.at[sl]], gx.at[sl], semg[b])
      pltpu.async_copy(py.at[idst.at[sl]], gy.at[sl], semg[b])
      pltpu.async_copy(pz.at[idst.at[sl]], gz.at[sl], semg[b])
      pltpu.async_copy(px.at[isrc.at[sl]], hx.at[sl], semg[b])
      pltpu.async_copy(py.at[isrc.at[sl]], hy.at[sl], semg[b])
      pltpu.async_copy(pz.at[isrc.at[sl]], hz.at[sl], semg[b])

  def drain_gat(b):
    ra, rb = rows_bufs[b]
    gx, gy, gz, hx, hy, hz = pos_bufs[b]
    for j in range(CH // 128):
      sl = pl.ds(j * 128, 128)
      pltpu.make_async_copy(dis_out.at[pl.ds(0, 128)], ra.at[sl],
                            semg[b]).wait()
      pltpu.make_async_copy(dis_out.at[pl.ds(0, 128)], rb.at[sl],
                            semg[b]).wait()
      for gbuf in (gx, gy, gz, hx, hy, hz):
        pltpu.make_async_copy(px.at[pl.ds(0, 128)], gbuf.at[sl],
                              semg[b]).wait()

  def compute_store(c, b):
    base = c * CH
    ra, rb = rows_bufs[b]
    gx, gy, gz, hx, hy, hz = pos_bufs[b]
    dirb = dir_bufs[b]
    pltpu.async_copy(ra, dis_out.at[pl.ds(base, CH)], sems[b])
    pltpu.async_copy(rb, bond_out.at[pl.ds(base, CH)], sems[b])

    def gbody(g, carry):
      rows = g * L + iota
      sl16 = pl.ds(g * L, L)
      dx = gx[sl16] - hx[sl16]
      dy = gy[sl16] - hy[sl16]
      dz = gz[sl16] - hz[sl16]
      s = dx * dx + dy * dy + dz * dz
      ib_ = lax.bitcast_convert_type(s, jnp.int32)
      y = lax.bitcast_convert_type(
          jnp.int32(0x5F3759DF) - lax.shift_right_logical(ib_, 1),
          jnp.float32)
      half = s * jnp.float32(0.5)
      for _ in range(3):
        y = y * (jnp.float32(1.5) - half * y * y)
      nrm = s * y
      inv = jnp.float32(1.0) / (nrm + jnp.float32(1e-8))
      flat = rows * 3
      plsc.store_scatter(dirb, [flat], dx * inv)
      plsc.store_scatter(dirb, [flat + 1], dy * inv)
      plsc.store_scatter(dirb, [flat + 2], dz * inv)
      return carry

    lax.fori_loop(0, CH // L, gbody, 0)
    pltpu.async_copy(dirb, dir_out.at[pl.ds(base * 3, CH * 3)], sems[b])

  def drain_st(b):
    ra, rb = rows_bufs[b]
    pltpu.make_async_copy(ra, dis_out.at[pl.ds(0, CH)], sems[b]).wait()
    pltpu.make_async_copy(rb, bond_out.at[pl.ds(0, CH)], sems[b]).wait()
    pltpu.make_async_copy(dir_bufs[b], dir_out.at[pl.ds(0, CH * 3)],
                          sems[b]).wait()

  @pl.when(cid == 0)
  def _degree():
    def zfill(i, carry):
      zb[pl.ds(i * L, L)] = jnp.zeros((L,), jnp.float32)
      return carry
    lax.fori_loop(0, DEG_SLICE // L, zfill, 0)
    off = tid * DEG_SLICE

    @pl.when(tid < NS - 1)
    def _():
      pltpu.sync_copy(zb, hist_sh.at[pl.ds(off, DEG_SLICE)])

    @pl.when(tid == NS - 1)
    def _():
      pltpu.sync_copy(zb.at[pl.ds(0, 2048)], hist_sh.at[pl.ds(off, 2048)])

    def ofill(i, carry):
      onesb[pl.ds(i * L, L)] = jnp.ones((L,), jnp.float32)
      return carry
    lax.fori_loop(0, 128 // L, ofill, 0)
    plsc.subcore_barrier()

    def dgroup(j, carry):
      g = tid + j * NS

      @pl.when(g < DEG_GROUPS)
      def _():
        pltpu.sync_copy(dstdeg2d.at[pl.ds(g * 16, 16)], dstb)
        for jj in range(16):
          pltpu.sync_copy(onesb, hist_sh.at[dstb.at[jj]], add=True)
      return carry

    lax.fori_loop(0, (DEG_GROUPS + NS - 1) // NS, dgroup, 0)
    plsc.subcore_barrier()

    @pl.when(tid < NS - 1)
    def _():
      pltpu.sync_copy(hist_sh.at[pl.ds(off, DEG_SLICE)],
                      deg_out.at[pl.ds(off, DEG_SLICE)])

    @pl.when(tid == NS - 1)
    def _():
      pltpu.sync_copy(hist_sh.at[pl.ds(off, 2048)],
                      deg_out.at[pl.ds(off, 2048)])

  @pl.when(cid == 1)
  def _nodes():
    def nchunk(j, carry):
      c = tid + j * NS

      @pl.when(c < NODE_FULL)
      def _():
        base = c * CH
        pltpu.sync_copy(atom1d.at[pl.ds(base, CH)], ia0)
        cps = []
        for jj in range(CH // 128):
          sl = pl.ds(jj * 128, 128)
          cps.append(pltpu.async_copy(atab.at[ia0.at[sl]], ra0.at[sl],
                                      semg0))
        for cp in cps:
          cp.wait()
        pltpu.sync_copy(ra0, node_out.at[pl.ds(base, CH)])

      @pl.when(c == NODE_FULL)
      def _():
        base = NODE_FULL * CH
        pltpu.sync_copy(atom1d.at[pl.ds(base, 128)], ia0.at[pl.ds(0, 128)])
        pltpu.async_copy(atab.at[ia0.at[pl.ds(0, 128)]],
                         ra0.at[pl.ds(0, 128)], semg0).wait()
        pltpu.sync_copy(ra0.at[pl.ds(0, NODE_TAIL)],
                        node_out.at[pl.ds(base, NODE_TAIL)])
      return carry

    lax.fori_loop(0, NODE_FULL // NS + 1, nchunk, 0)

  fire_idx(wid, 0)
  fire_idx(wid + NW, 1)
  drain_idx(0)
  fire_gat(0)

  def pair(jj, carry):
    for b in (0, 1):
      c = wid + (2 * jj + b) * NW
      bo = 1 - b

      @pl.when(c < NCH)
      def _():
        drain_gat(b)

        @pl.when(c + NW < NCH)
        def _():
          drain_idx(bo)
          if b == 0:
            @pl.when(jj > 0)
            def _():
              drain_st(bo)
          else:
            drain_st(bo)
          fire_gat(bo)

        @pl.when(c + 2 * NW < NCH)
        def _():
          fire_idx(c + 2 * NW, b)

        compute_store(c, b)
    return carry

  lax.fori_loop(0, NPAIR, pair, 0)
  drain_st(0)
  drain_st(1)


_sc_call = pl.kernel(
    _sc_body,
    out_type=[
        jax.ShapeDtypeStruct((N_NODES, EMB), jnp.float32),
        jax.ShapeDtypeStruct((N_EDGES, EMB), jnp.float32),
        jax.ShapeDtypeStruct((N_EDGES, EMB), jnp.float32),
        jax.ShapeDtypeStruct((HIST_PAD,), jnp.float32),
        jax.ShapeDtypeStruct((N_EDGES * 3,), jnp.float32),
    ],
    mesh=plsc.VectorSubcoreMesh(core_axis_name="c", subcore_axis_name="s"),
    compiler_params=pltpu.CompilerParams(use_tc_tiling_on_sc=False,
                                         needs_layout_passes=False),
    scratch_types=(
        [pltpu.VMEM((CH,), jnp.int32) for _ in range(8)]
        + [pltpu.VMEM((CH, EMB), jnp.float32) for _ in range(4)]
        + [pltpu.VMEM((CH,), jnp.float32) for _ in range(12)]
        + [pltpu.VMEM((CH * 3,), jnp.float32) for _ in range(2)]
        + [pltpu.VMEM((128,), jnp.float32),
           pltpu.VMEM((16, 128), jnp.int32),
           pltpu.VMEM((DEG_SLICE,), jnp.float32),
           pltpu.VMEM_SHARED((HIST_PAD,), jnp.float32)]
        + [pltpu.SemaphoreType.DMA for _ in range(6)]
    ),
)


def kernel(atom_types, edge_index, bond_types, dist_bins, pos,
           atom_table, bond_table, dist_table):
  atom1d = jnp.pad(atom_types.astype(jnp.int32), (0, NODE_PAD - N_NODES))
  bins1d = dist_bins.astype(jnp.int32)
  bonds1d = bond_types.astype(jnp.int32)
  src1d = edge_index[0].astype(jnp.int32)
  dst1d = edge_index[1].astype(jnp.int32)
  dstdeg2d = jnp.pad(dst1d, (0, DEG_ROWS * 128 - N_EDGES),
                     constant_values=N_NODES).reshape(DEG_ROWS, 128)
  px = pos[:, 0]
  py = pos[:, 1]
  pz = pos[:, 2]
  node_feat, edge_dis, edge_bond, degree, edges_dir = _sc_call(
      atom1d, bins1d, bonds1d, src1d, dst1d, dstdeg2d, px, py, pz,
      atom_table, bond_table, dist_table)
  return (node_feat, edge_dis, edge_bond, degree[:N_NODES],
          edges_dir.reshape(N_EDGES, 3))


# speedup vs baseline: 1.0024x; 1.0024x over previous
import jax
import jax.numpy as jnp
from jax import lax
from jax.experimental import pallas as pl
from jax.experimental.pallas import tpu as pltpu
from jax.experimental.pallas import tpu_sc as plsc

N_NODES = 50000
N_EDGES = 800000
EMB = 64
NC, NS = 2, 16
NW = NC * NS
L = 16

CH = 256
NCH = N_EDGES // CH
NPAIR = 49

NODE_FULL = N_NODES // CH
NODE_TAIL = N_NODES - NODE_FULL * CH
NODE_PAD = 50048

DEG_ROWS = 6256
DEG_GROUPS = DEG_ROWS // 16
HIST_PAD = 50048
DEG_SLICE = 3200


def _sc_body(atom1d, bins1d, bonds1d, src1d, dst1d, dstdeg2d, px, py, pz,
             atab, btab, dtab,
             node_out, dis_out, bond_out, deg_out, dir_out,
             ia0, ib0, is0, id0, ia1, ib1, is1, id1,
             ra0, rb0, ra1, rb1,
             px0, py0, pz0, qx0, qy0, qz0,
             px1, py1, pz1, qx1, qy1, qz1,
             dir0, dir1,
             onesb, dstb, zb, hist_sh,
             semi0, semi1, semg0, semg1, sems0, sems1, semd):
  cid = lax.axis_index("c")
  tid = lax.axis_index("s")
  wid = tid * NC + cid
  iota = lax.iota(jnp.int32, L)

  idx_bufs = ((ia0, ib0, is0, id0), (ia1, ib1, is1, id1))
  rows_bufs = ((ra0, rb0), (ra1, rb1))
  pos_bufs = ((px0, py0, pz0, qx0, qy0, qz0),
              (px1, py1, pz1, qx1, qy1, qz1))
  dir_bufs = (dir0, dir1)
  semi = (semi0, semi1)
  semg = (semg0, semg1)
  sems = (sems0, sems1)
  idx_srcs = (bins1d, bonds1d, src1d, dst1d)

  def fire_idx(c, b):
    base = c * CH
    for buf, src in zip(idx_bufs[b], idx_srcs):
      pltpu.async_copy(src.at[pl.ds(base, CH)], buf, semi[b])

  def drain_idx(b):
    for buf, src in zip(idx_bufs[b], idx_srcs):
      pltpu.make_async_copy(src.at[pl.ds(0, CH)], buf, semi[b]).wait()

  def fire_gat(b):
    ia, ib, isrc, idst = idx_bufs[b]
    ra, rb = rows_bufs[b]
    gx, gy, gz, hx, hy, hz = pos_bufs[b]
    for j in range(CH // 128):
      sl = pl.ds(j * 128, 128)
      pltpu.async_copy(dtab.at[ia.at[sl]], ra.at[sl], semg[b])
      pltpu.async_copy(btab.at[ib.at[sl]], rb.at[sl], semg[b])
      pltpu.async_copy(px.at[idst.at[sl]], gx.at[sl], semg[b])
      pltpu.async_copy(py.at[idst.at[sl]], gy.at[sl], semg[b])
      pltpu.async_copy(pz.at[idst.at[sl]], gz.at[sl], semg[b])
      pltpu.async_copy(px.at[isrc.at[sl]], hx.at[sl], semg[b])
      pltpu.async_copy(py.at[isrc.at[sl]], hy.at[sl], semg[b])
      pltpu.async_copy(pz.at[isrc.at[sl]], hz.at[sl], semg[b])

  def drain_gat(b):
    ra, rb = rows_bufs[b]
    gx, gy, gz, hx, hy, hz = pos_bufs[b]
    for j in range(CH // 128):
      sl = pl.ds(j * 128, 128)
      pltpu.make_async_copy(dis_out.at[pl.ds(0, 128)], ra.at[sl],
                            semg[b]).wait()
      pltpu.make_async_copy(dis_out.at[pl.ds(0, 128)], rb.at[sl],
                            semg[b]).wait()
      for gbuf in (gx, gy, gz, hx, hy, hz):
        pltpu.make_async_copy(px.at[pl.ds(0, 128)], gbuf.at[sl],
                              semg[b]).wait()

  def compute_store(c, b):
    base = c * CH
    ra, rb = rows_bufs[b]
    gx, gy, gz, hx, hy, hz = pos_bufs[b]
    dirb = dir_bufs[b]
    pltpu.async_copy(ra, dis_out.at[pl.ds(base, CH)], sems[b])
    pltpu.async_copy(rb, bond_out.at[pl.ds(base, CH)], sems[b])

    def gbody(g, carry):
      rows = g * L + iota
      sl16 = pl.ds(g * L, L)
      dx = gx[sl16] - hx[sl16]
      dy = gy[sl16] - hy[sl16]
      dz = gz[sl16] - hz[sl16]
      s = dx * dx + dy * dy + dz * dz
      ib_ = lax.bitcast_convert_type(s, jnp.int32)
      y = lax.bitcast_convert_type(
          jnp.int32(0x5F3759DF) - lax.shift_right_logical(ib_, 1),
          jnp.float32)
      half = s * jnp.float32(0.5)
      for _ in range(3):
        y = y * (jnp.float32(1.5) - half * y * y)
      nrm = s * y
      inv = jnp.float32(1.0) / (nrm + jnp.float32(1e-8))
      flat = rows * 3
      plsc.store_scatter(dirb, [flat], dx * inv)
      plsc.store_scatter(dirb, [flat + 1], dy * inv)
      plsc.store_scatter(dirb, [flat + 2], dz * inv)
      return carry

    lax.fori_loop(0, CH // L, gbody, 0)
    pltpu.async_copy(dirb, dir_out.at[pl.ds(base * 3, CH * 3)], sems[b])

  def drain_st(b):
    ra, rb = rows_bufs[b]
    pltpu.make_async_copy(ra, dis_out.at[pl.ds(0, CH)], sems[b]).wait()
    pltpu.make_async_copy(rb, bond_out.at[pl.ds(0, CH)], sems[b]).wait()
    pltpu.make_async_copy(dir_bufs[b], dir_out.at[pl.ds(0, CH * 3)],
                          sems[b]).wait()

  @pl.when(cid == 0)
  def _degree():
    def zfill(i, carry):
      zb[pl.ds(i * L, L)] = jnp.zeros((L,), jnp.float32)
      return carry
    lax.fori_loop(0, DEG_SLICE // L, zfill, 0)
    off = tid * DEG_SLICE

    @pl.when(tid < NS - 1)
    def _():
      pltpu.sync_copy(zb, hist_sh.at[pl.ds(off, DEG_SLICE)])

    @pl.when(tid == NS - 1)
    def _():
      pltpu.sync_copy(zb.at[pl.ds(0, 2048)], hist_sh.at[pl.ds(off, 2048)])

    def ofill(i, carry):
      onesb[pl.ds(i * L, L)] = jnp.ones((L,), jnp.float32)
      return carry
    lax.fori_loop(0, 128 // L, ofill, 0)
    plsc.subcore_barrier()

    def dgroup(j, carry):
      g = tid + j * NS

      @pl.when(g < DEG_GROUPS)
      def _():
        pltpu.sync_copy(dstdeg2d.at[pl.ds(g * 16, 16)], dstb)
        for jj in range(16):
          pltpu.async_copy(onesb, hist_sh.at[dstb.at[jj]], semd, add=True)
        for jj in range(16):
          pltpu.make_async_copy(px.at[pl.ds(0, 128)], onesb, semd).wait()
      return carry

    lax.fori_loop(0, (DEG_GROUPS + NS - 1) // NS, dgroup, 0)
    plsc.subcore_barrier()

    @pl.when(tid < NS - 1)
    def _():
      pltpu.sync_copy(hist_sh.at[pl.ds(off, DEG_SLICE)],
                      deg_out.at[pl.ds(off, DEG_SLICE)])

    @pl.when(tid == NS - 1)
    def _():
      pltpu.sync_copy(hist_sh.at[pl.ds(off, 2048)],
                      deg_out.at[pl.ds(off, 2048)])

  @pl.when(cid == 1)
  def _nodes():
    def nchunk(j, carry):
      c = tid + j * NS

      @pl.when(c < NODE_FULL)
      def _():
        base = c * CH
        pltpu.sync_copy(atom1d.at[pl.ds(base, CH)], ia0)
        cps = []
        for jj in range(CH // 128):
          sl = pl.ds(jj * 128, 128)
          cps.append(pltpu.async_copy(atab.at[ia0.at[sl]], ra0.at[sl],
                                      semg0))
        for cp in cps:
          cp.wait()
        pltpu.sync_copy(ra0, node_out.at[pl.ds(base, CH)])

      @pl.when(c == NODE_FULL)
      def _():
        base = NODE_FULL * CH
        pltpu.sync_copy(atom1d.at[pl.ds(base, 128)], ia0.at[pl.ds(0, 128)])
        pltpu.async_copy(atab.at[ia0.at[pl.ds(0, 128)]],
                         ra0.at[pl.ds(0, 128)], semg0).wait()
        pltpu.sync_copy(ra0.at[pl.ds(0, NODE_TAIL)],
                        node_out.at[pl.ds(base, NODE_TAIL)])
      return carry

    lax.fori_loop(0, NODE_FULL // NS + 1, nchunk, 0)

  fire_idx(wid, 0)
  fire_idx(wid + NW, 1)
  drain_idx(0)
  fire_gat(0)

  def pair(jj, carry):
    for b in (0, 1):
      c = wid + (2 * jj + b) * NW
      bo = 1 - b

      @pl.when(c < NCH)
      def _():
        drain_gat(b)

        @pl.when(c + NW < NCH)
        def _():
          drain_idx(bo)
          if b == 0:
            @pl.when(jj > 0)
            def _():
              drain_st(bo)
          else:
            drain_st(bo)
          fire_gat(bo)

        @pl.when(c + 2 * NW < NCH)
        def _():
          fire_idx(c + 2 * NW, b)

        compute_store(c, b)
    return carry

  lax.fori_loop(0, NPAIR, pair, 0)
  drain_st(0)
  drain_st(1)


_sc_call = pl.kernel(
    _sc_body,
    out_type=[
        jax.ShapeDtypeStruct((N_NODES, EMB), jnp.float32),
        jax.ShapeDtypeStruct((N_EDGES, EMB), jnp.float32),
        jax.ShapeDtypeStruct((N_EDGES, EMB), jnp.float32),
        jax.ShapeDtypeStruct((HIST_PAD,), jnp.float32),
        jax.ShapeDtypeStruct((N_EDGES * 3,), jnp.float32),
    ],
    mesh=plsc.VectorSubcoreMesh(core_axis_name="c", subcore_axis_name="s"),
    compiler_params=pltpu.CompilerParams(use_tc_tiling_on_sc=False,
                                         needs_layout_passes=False),
    scratch_types=(
        [pltpu.VMEM((CH,), jnp.int32) for _ in range(8)]
        + [pltpu.VMEM((CH, EMB), jnp.float32) for _ in range(4)]
        + [pltpu.VMEM((CH,), jnp.float32) for _ in range(12)]
        + [pltpu.VMEM((CH * 3,), jnp.float32) for _ in range(2)]
        + [pltpu.VMEM((128,), jnp.float32),
           pltpu.VMEM((16, 128), jnp.int32),
           pltpu.VMEM((DEG_SLICE,), jnp.float32),
           pltpu.VMEM_SHARED((HIST_PAD,), jnp.float32)]
        + [pltpu.SemaphoreType.DMA for _ in range(7)]
    ),
)


def kernel(atom_types, edge_index, bond_types, dist_bins, pos,
           atom_table, bond_table, dist_table):
  atom1d = jnp.pad(atom_types.astype(jnp.int32), (0, NODE_PAD - N_NODES))
  bins1d = dist_bins.astype(jnp.int32)
  bonds1d = bond_types.astype(jnp.int32)
  src1d = edge_index[0].astype(jnp.int32)
  dst1d = edge_index[1].astype(jnp.int32)
  dstdeg2d = jnp.pad(dst1d, (0, DEG_ROWS * 128 - N_EDGES),
                     constant_values=N_NODES).reshape(DEG_ROWS, 128)
  px = pos[:, 0]
  py = pos[:, 1]
  pz = pos[:, 2]
  node_feat, edge_dis, edge_bond, degree, edges_dir = _sc_call(
      atom1d, bins1d, bonds1d, src1d, dst1d, dstdeg2d, px, py, pz,
      atom_table, bond_table, dist_table)
  return (node_feat, edge_dis, edge_bond, degree[:N_NODES],
          edges_dir.reshape(N_EDGES, 3))


# speedup vs baseline: 2.4429x; 2.4371x over previous
import jax
import jax.numpy as jnp
from jax import lax
from jax.experimental import pallas as pl
from jax.experimental.pallas import tpu as pltpu
from jax.experimental.pallas import tpu_sc as plsc

N_NODES = 50000
N_EDGES = 800000
EMB = 64
NC, NS = 2, 16
NW = NC * NS
L = 16

CH = 256
NCH = N_EDGES // CH
NPAIR = 49

NODE_FULL = N_NODES // CH
NODE_TAIL = N_NODES - NODE_FULL * CH
NODE_PAD = 50048

DEG_ROWS = 6256
DEG_GROUPS = DEG_ROWS // 16
HIST_PAD = 50048
DEG_SLICE = 3200


def _sc_body(atom1d, bins1d, bonds1d, src1d, dst1d, dstdeg2d, px, py, pz,
             atab, btab, dtab,
             node_out, dis_out, bond_out, deg_out, dir_out,
             ia0, ib0, is0, id0, ia1, ib1, is1, id1,
             ra0, rb0, ra1, rb1,
             px0, py0, pz0, qx0, qy0, qz0,
             px1, py1, pz1, qx1, qy1, qz1,
             dir0, dir1,
             onesb, dstb, zb, hist_sh,
             asp, dsp, bsp, pxs, pys, pzs,
             semi0, semi1, semg0, semg1, sems0, sems1, semd):
  cid = lax.axis_index("c")
  tid = lax.axis_index("s")
  wid = tid * NC + cid
  iota = lax.iota(jnp.int32, L)

  idx_bufs = ((ia0, ib0, is0, id0), (ia1, ib1, is1, id1))
  rows_bufs = ((ra0, rb0), (ra1, rb1))
  pos_bufs = ((px0, py0, pz0, qx0, qy0, qz0),
              (px1, py1, pz1, qx1, qy1, qz1))
  dir_bufs = (dir0, dir1)
  semi = (semi0, semi1)
  semg = (semg0, semg1)
  sems = (sems0, sems1)
  idx_srcs = (bins1d, bonds1d, src1d, dst1d)

  def fire_idx(c, b):
    base = c * CH
    for buf, src in zip(idx_bufs[b], idx_srcs):
      pltpu.async_copy(src.at[pl.ds(base, CH)], buf, semi[b])

  def drain_idx(b):
    for buf, src in zip(idx_bufs[b], idx_srcs):
      pltpu.make_async_copy(src.at[pl.ds(0, CH)], buf, semi[b]).wait()

  def fire_gat(b):
    ia, ib, isrc, idst = idx_bufs[b]
    ra, rb = rows_bufs[b]
    gx, gy, gz, hx, hy, hz = pos_bufs[b]
    for j in range(CH // 128):
      sl = pl.ds(j * 128, 128)
      pltpu.async_copy(dsp.at[ia.at[sl]], ra.at[sl], semg[b])
      pltpu.async_copy(bsp.at[ib.at[sl]], rb.at[sl], semg[b])
      pltpu.async_copy(pxs.at[idst.at[sl]], gx.at[sl], semg[b])
      pltpu.async_copy(pys.at[idst.at[sl]], gy.at[sl], semg[b])
      pltpu.async_copy(pzs.at[idst.at[sl]], gz.at[sl], semg[b])
      pltpu.async_copy(pxs.at[isrc.at[sl]], hx.at[sl], semg[b])
      pltpu.async_copy(pys.at[isrc.at[sl]], hy.at[sl], semg[b])
      pltpu.async_copy(pzs.at[isrc.at[sl]], hz.at[sl], semg[b])

  def drain_gat(b):
    ra, rb = rows_bufs[b]
    gx, gy, gz, hx, hy, hz = pos_bufs[b]
    for j in range(CH // 128):
      sl = pl.ds(j * 128, 128)
      pltpu.make_async_copy(dis_out.at[pl.ds(0, 128)], ra.at[sl],
                            semg[b]).wait()
      pltpu.make_async_copy(dis_out.at[pl.ds(0, 128)], rb.at[sl],
                            semg[b]).wait()
      for gbuf in (gx, gy, gz, hx, hy, hz):
        pltpu.make_async_copy(px.at[pl.ds(0, 128)], gbuf.at[sl],
                              semg[b]).wait()

  def compute_store(c, b):
    base = c * CH
    ra, rb = rows_bufs[b]
    gx, gy, gz, hx, hy, hz = pos_bufs[b]
    dirb = dir_bufs[b]
    pltpu.async_copy(ra, dis_out.at[pl.ds(base, CH)], sems[b])
    pltpu.async_copy(rb, bond_out.at[pl.ds(base, CH)], sems[b])

    def gbody(g, carry):
      rows = g * L + iota
      sl16 = pl.ds(g * L, L)
      dx = gx[sl16] - hx[sl16]
      dy = gy[sl16] - hy[sl16]
      dz = gz[sl16] - hz[sl16]
      s = dx * dx + dy * dy + dz * dz
      ib_ = lax.bitcast_convert_type(s, jnp.int32)
      y = lax.bitcast_convert_type(
          jnp.int32(0x5F3759DF) - lax.shift_right_logical(ib_, 1),
          jnp.float32)
      half = s * jnp.float32(0.5)
      for _ in range(3):
        y = y * (jnp.float32(1.5) - half * y * y)
      nrm = s * y
      inv = jnp.float32(1.0) / (nrm + jnp.float32(1e-8))
      flat = rows * 3
      plsc.store_scatter(dirb, [flat], dx * inv)
      plsc.store_scatter(dirb, [flat + 1], dy * inv)
      plsc.store_scatter(dirb, [flat + 2], dz * inv)
      return carry

    lax.fori_loop(0, CH // L, gbody, 0)
    pltpu.async_copy(dirb, dir_out.at[pl.ds(base * 3, CH * 3)], sems[b])

  def drain_st(b):
    ra, rb = rows_bufs[b]
    pltpu.make_async_copy(ra, dis_out.at[pl.ds(0, CH)], sems[b]).wait()
    pltpu.make_async_copy(rb, bond_out.at[pl.ds(0, CH)], sems[b]).wait()
    pltpu.make_async_copy(dir_bufs[b], dir_out.at[pl.ds(0, CH * 3)],
                          sems[b]).wait()

  @pl.when(tid == 0)
  def _():
    pltpu.sync_copy(atab, asp)
    pltpu.sync_copy(dtab, dsp)
    pltpu.sync_copy(btab, bsp)

  @pl.when(tid == 1)
  def _():
    pltpu.sync_copy(px, pxs)

  @pl.when(tid == 2)
  def _():
    pltpu.sync_copy(py, pys)

  @pl.when(tid == 3)
  def _():
    pltpu.sync_copy(pz, pzs)

  plsc.subcore_barrier()

  @pl.when(cid == 0)
  def _degree():
    def zfill(i, carry):
      zb[pl.ds(i * L, L)] = jnp.zeros((L,), jnp.float32)
      return carry
    lax.fori_loop(0, DEG_SLICE // L, zfill, 0)
    off = tid * DEG_SLICE

    @pl.when(tid < NS - 1)
    def _():
      pltpu.sync_copy(zb, hist_sh.at[pl.ds(off, DEG_SLICE)])

    @pl.when(tid == NS - 1)
    def _():
      pltpu.sync_copy(zb.at[pl.ds(0, 2048)], hist_sh.at[pl.ds(off, 2048)])

    def ofill(i, carry):
      onesb[pl.ds(i * L, L)] = jnp.ones((L,), jnp.float32)
      return carry
    lax.fori_loop(0, 128 // L, ofill, 0)
    plsc.subcore_barrier()

    def dgroup(j, carry):
      g = tid + j * NS

      @pl.when(g < DEG_GROUPS)
      def _():
        pltpu.sync_copy(dstdeg2d.at[pl.ds(g * 16, 16)], dstb)
        for jj in range(16):
          pltpu.async_copy(onesb, hist_sh.at[dstb.at[jj]], semd, add=True)
        for jj in range(16):
          pltpu.make_async_copy(px.at[pl.ds(0, 128)], onesb, semd).wait()
      return carry

    lax.fori_loop(0, (DEG_GROUPS + NS - 1) // NS, dgroup, 0)
    plsc.subcore_barrier()

    @pl.when(tid < NS - 1)
    def _():
      pltpu.sync_copy(hist_sh.at[pl.ds(off, DEG_SLICE)],
                      deg_out.at[pl.ds(off, DEG_SLICE)])

    @pl.when(tid == NS - 1)
    def _():
      pltpu.sync_copy(hist_sh.at[pl.ds(off, 2048)],
                      deg_out.at[pl.ds(off, 2048)])

  @pl.when(cid == 1)
  def _nodes():
    def nchunk(j, carry):
      c = tid + j * NS

      @pl.when(c < NODE_FULL)
      def _():
        base = c * CH
        pltpu.sync_copy(atom1d.at[pl.ds(base, CH)], ia0)
        cps = []
        for jj in range(CH // 128):
          sl = pl.ds(jj * 128, 128)
          cps.append(pltpu.async_copy(asp.at[ia0.at[sl]], ra0.at[sl],
                                      semg0))
        for cp in cps:
          cp.wait()
        pltpu.sync_copy(ra0, node_out.at[pl.ds(base, CH)])

      @pl.when(c == NODE_FULL)
      def _():
        base = NODE_FULL * CH
        pltpu.sync_copy(atom1d.at[pl.ds(base, 128)], ia0.at[pl.ds(0, 128)])
        pltpu.async_copy(asp.at[ia0.at[pl.ds(0, 128)]],
                         ra0.at[pl.ds(0, 128)], semg0).wait()
        pltpu.sync_copy(ra0.at[pl.ds(0, NODE_TAIL)],
                        node_out.at[pl.ds(base, NODE_TAIL)])
      return carry

    lax.fori_loop(0, NODE_FULL // NS + 1, nchunk, 0)

  fire_idx(wid, 0)
  fire_idx(wid + NW, 1)
  drain_idx(0)
  fire_gat(0)

  def pair(jj, carry):
    for b in (0, 1):
      c = wid + (2 * jj + b) * NW
      bo = 1 - b

      @pl.when(c < NCH)
      def _():
        drain_gat(b)

        @pl.when(c + NW < NCH)
        def _():
          drain_idx(bo)
          if b == 0:
            @pl.when(jj > 0)
            def _():
              drain_st(bo)
          else:
            drain_st(bo)
          fire_gat(bo)

        @pl.when(c + 2 * NW < NCH)
        def _():
          fire_idx(c + 2 * NW, b)

        compute_store(c, b)
    return carry

  lax.fori_loop(0, NPAIR, pair, 0)
  drain_st(0)
  drain_st(1)


_sc_call = pl.kernel(
    _sc_body,
    out_type=[
        jax.ShapeDtypeStruct((N_NODES, EMB), jnp.float32),
        jax.ShapeDtypeStruct((N_EDGES, EMB), jnp.float32),
        jax.ShapeDtypeStruct((N_EDGES, EMB), jnp.float32),
        jax.ShapeDtypeStruct((HIST_PAD,), jnp.float32),
        jax.ShapeDtypeStruct((N_EDGES * 3,), jnp.float32),
    ],
    mesh=plsc.VectorSubcoreMesh(core_axis_name="c", subcore_axis_name="s"),
    compiler_params=pltpu.CompilerParams(use_tc_tiling_on_sc=False,
                                         needs_layout_passes=False),
    scratch_types=(
        [pltpu.VMEM((CH,), jnp.int32) for _ in range(8)]
        + [pltpu.VMEM((CH, EMB), jnp.float32) for _ in range(4)]
        + [pltpu.VMEM((CH,), jnp.float32) for _ in range(12)]
        + [pltpu.VMEM((CH * 3,), jnp.float32) for _ in range(2)]
        + [pltpu.VMEM((128,), jnp.float32),
           pltpu.VMEM((16, 128), jnp.int32),
           pltpu.VMEM((DEG_SLICE,), jnp.float32),
           pltpu.VMEM_SHARED((HIST_PAD,), jnp.float32),
           pltpu.VMEM_SHARED((119, EMB), jnp.float32),
           pltpu.VMEM_SHARED((64, EMB), jnp.float32),
           pltpu.VMEM_SHARED((22, EMB), jnp.float32),
           pltpu.VMEM_SHARED((N_NODES,), jnp.float32),
           pltpu.VMEM_SHARED((N_NODES,), jnp.float32),
           pltpu.VMEM_SHARED((N_NODES,), jnp.float32)]
        + [pltpu.SemaphoreType.DMA for _ in range(7)]
    ),
)


def kernel(atom_types, edge_index, bond_types, dist_bins, pos,
           atom_table, bond_table, dist_table):
  atom1d = jnp.pad(atom_types.astype(jnp.int32), (0, NODE_PAD - N_NODES))
  bins1d = dist_bins.astype(jnp.int32)
  bonds1d = bond_types.astype(jnp.int32)
  src1d = edge_index[0].astype(jnp.int32)
  dst1d = edge_index[1].astype(jnp.int32)
  dstdeg2d = jnp.pad(dst1d, (0, DEG_ROWS * 128 - N_EDGES),
                     constant_values=N_NODES).reshape(DEG_ROWS, 128)
  px = pos[:, 0]
  py = pos[:, 1]
  pz = pos[:, 2]
  node_feat, edge_dis, edge_bond, degree, edges_dir = _sc_call(
      atom1d, bins1d, bonds1d, src1d, dst1d, dstdeg2d, px, py, pz,
      atom_table, bond_table, dist_table)
  return (node_feat, edge_dis, edge_bond, degree[:N_NODES],
          edges_dir.reshape(N_EDGES, 3))
